# Initial kernel scaffold; baseline (speedup 1.0000x reference)
import jax
import jax.numpy as jnp
from jax.experimental import pallas as pl


def kernel(node_wid_list, node_child_adjacency_graph, node_edge_adjacency_graph, edge_node_adjacency_graph, scope, root_scope, params):
    raise NotImplementedError("write your pallas kernel here")


# speedup vs baseline: 1.3443x; 1.3443x over previous
import functools

import jax
import jax.numpy as jnp
from jax import lax
from jax.experimental import pallas as pl
from jax.experimental.pallas import tpu as pltpu
from jax.experimental.pallas import tpu_sc as plsc

H = 128
MAXN = 8
NN = 10001
NE = 160000
NMOL = 256
SEG = 625
NC = 2
NS = 16
NW = NC * NS
NPAD = 10240
EPAD = 163840
DEAD = NE
NPW = NPAD // NW
EPW = EPAD // NW
ECH = 512
NBLK = 1280
EBLK = 5120
PBLK = 8 * SEG


def _sc_mesh():
    return plsc.VectorSubcoreMesh(core_axis_name="c", subcore_axis_name="s")


def _wid():
    return lax.axis_index("s") * NC + lax.axis_index("c")


_SC_CACHE = {}


def _sc_kernels():
    if "embed" in _SC_CACHE:
        return _SC_CACHE

    @functools.partial(
        pl.kernel, mesh=_sc_mesh(),
        out_type=jax.ShapeDtypeStruct((NPAD, H), jnp.float32),
        scratch_types=[pltpu.VMEM((NPW,), jnp.int32),
                       pltpu.VMEM((NPW, H), jnp.float32),
                       pltpu.SemaphoreType.DMA])
    def _embed_k(idx_hbm, tab_hbm, out_hbm, idx_v, rows_v, sem):
        base = _wid() * NPW
        pltpu.sync_copy(idx_hbm.at[pl.ds(base, NPW)], idx_v)
        pltpu.async_copy(tab_hbm.at[idx_v], rows_v, sem).wait()
        pltpu.sync_copy(rows_v, out_hbm.at[pl.ds(base, NPW)])

    @functools.partial(
        pl.kernel, mesh=_sc_mesh(),
        out_type=(jax.ShapeDtypeStruct((EPAD, H), jnp.float32),
                  jax.ShapeDtypeStruct((EPAD, H), jnp.float32)),
        scratch_types=[pltpu.VMEM((ECH,), jnp.int32),
                       pltpu.VMEM((ECH, H), jnp.float32),
                       pltpu.SemaphoreType.DMA])
    def _egather_k(srcp, dstp, tab_b, tab_c, out_b, out_c, idx_v, rows_v, sem):
        base = _wid() * EPW
        for c in range(EPW // ECH):
            off = base + c * ECH
            pltpu.sync_copy(srcp.at[pl.ds(off, ECH)], idx_v)
            pltpu.async_copy(tab_b.at[idx_v], rows_v, sem).wait()
            pltpu.sync_copy(rows_v, out_b.at[pl.ds(off, ECH)])
            pltpu.sync_copy(dstp.at[pl.ds(off, ECH)], idx_v)
            pltpu.async_copy(tab_c.at[idx_v], rows_v, sem).wait()
            pltpu.sync_copy(rows_v, out_c.at[pl.ds(off, ECH)])

    @functools.partial(
        pl.kernel, mesh=_sc_mesh(),
        out_type=(jax.ShapeDtypeStruct((MAXN, NPAD, H), jnp.float32),
                  jax.ShapeDtypeStruct((MAXN, NPAD, H), jnp.float32)),
        scratch_types=[pltpu.VMEM((NPW,), jnp.int32),
                       pltpu.VMEM((NPW, H), jnp.float32),
                       pltpu.SemaphoreType.DMA])
    def _ngather_k(idx_g, idx_c, etab, vtab, out_g, out_w, idx_v, rows_v, sem):
        base = _wid() * NPW
        for j in range(MAXN):
            pltpu.sync_copy(idx_g.at[pl.ds(j * NPAD + base, NPW)], idx_v)
            pltpu.async_copy(etab.at[idx_v], rows_v, sem).wait()
            pltpu.sync_copy(rows_v, out_g.at[j, pl.ds(base, NPW)])
            pltpu.sync_copy(idx_c.at[pl.ds(j * NPAD + base, NPW)], idx_v)
            pltpu.async_copy(vtab.at[idx_v], rows_v, sem).wait()
            pltpu.sync_copy(rows_v, out_w.at[j, pl.ds(base, NPW)])

    _SC_CACHE.update(embed=_embed_k, egather=_egather_k, ngather=_ngather_k)
    return _SC_CACHE


def _proj_body(x_ref, w_ref, b_ref, *outs):
    y = jnp.dot(x_ref[...], w_ref[...], preferred_element_type=jnp.float32)
    y = y + b_ref[...]
    for i, o in enumerate(outs):
        o[...] = y[:, i * H:(i + 1) * H]


def _node_body(ge_ref, wv_ref, xu_ref, w_ref, b_ref, *outs):
    acc = jax.nn.sigmoid(ge_ref[0]) * wv_ref[0]
    for j in range(1, MAXN):
        acc = acc + jax.nn.sigmoid(ge_ref[j]) * wv_ref[j]
    row = lax.broadcasted_iota(jnp.int32, (NBLK, 1), 0) + pl.program_id(0) * NBLK
    ok = (row >= 1) & (row < NN)
    x = jnp.where(ok, jax.nn.relu(xu_ref[...] + acc), 0.0)
    y = jnp.dot(x, w_ref[...], preferred_element_type=jnp.float32) + b_ref[...]
    for i, o in enumerate(outs):
        o[...] = y[:, i * H:(i + 1) * H]


def _edge0_body(xbs_ref, xcd_ref, b_ref, out_ref):
    row = lax.broadcasted_iota(jnp.int32, (EBLK, 1), 0) + pl.program_id(0) * EBLK
    ok = row < NE
    y = jax.nn.relu(xbs_ref[...] + xcd_ref[...] + b_ref[...])
    out_ref[...] = jnp.where(ok, y, 0.0)


def _edge1_body(e_ref, xbs_ref, xcd_ref, aw_ref, ab_ref, out_ref):
    row = lax.broadcasted_iota(jnp.int32, (EBLK, 1), 0) + pl.program_id(0) * EBLK
    ok = row < NE
    y = jnp.dot(e_ref[...], aw_ref[...], preferred_element_type=jnp.float32)
    y = jax.nn.relu(y + ab_ref[...] + xbs_ref[...] + xcd_ref[...])
    out_ref[...] = jnp.where(ok, y, 0.0)


def _pool_body(e_ref, xvs_ref, xwd_ref, wu_ref, bu_ref, wa_ref, ba_ref, out_ref):
    e = e_ref[...]
    syn = jnp.dot(e, wu_ref[...], preferred_element_type=jnp.float32)
    syn = syn + bu_ref[...] + xvs_ref[...] + xwd_ref[...]
    g = jax.nn.sigmoid(syn)
    vals = g * (jnp.dot(e, wa_ref[...], preferred_element_type=jnp.float32)
                + ba_ref[...])
    rows = [jnp.sum(vals[m * SEG:(m + 1) * SEG], axis=0, keepdims=True)
            for m in range(8)]
    out_ref[...] = jnp.concatenate(rows, axis=0)


def _bspec(shape, imap):
    return pl.BlockSpec(shape, imap)


def _proj(x, w, b, nout):
    return pl.pallas_call(
        _proj_body,
        grid=(NPAD // NBLK,),
        in_specs=[_bspec((NBLK, H), lambda k: (k, 0)),
                  _bspec((H, nout * H), lambda k: (0, 0)),
                  _bspec((1, nout * H), lambda k: (0, 0))],
        out_specs=[_bspec((NBLK, H), lambda k: (k, 0))] * nout,
        out_shape=[jax.ShapeDtypeStruct((NPAD, H), jnp.float32)] * nout,
    )(x, w, b)


def _node_update(ge, wv, xu, w, b, nout):
    return pl.pallas_call(
        _node_body,
        grid=(NPAD // NBLK,),
        in_specs=[_bspec((MAXN, NBLK, H), lambda k: (0, k, 0)),
                  _bspec((MAXN, NBLK, H), lambda k: (0, k, 0)),
                  _bspec((NBLK, H), lambda k: (k, 0)),
                  _bspec((H, nout * H), lambda k: (0, 0)),
                  _bspec((1, nout * H), lambda k: (0, 0))],
        out_specs=[_bspec((NBLK, H), lambda k: (k, 0))] * nout,
        out_shape=[jax.ShapeDtypeStruct((NPAD, H), jnp.float32)] * nout,
    )(ge, wv, xu, w, b)


def _edge0(xbs, xcd, b):
    return pl.pallas_call(
        _edge0_body,
        grid=(EPAD // EBLK,),
        in_specs=[_bspec((EBLK, H), lambda k: (k, 0)),
                  _bspec((EBLK, H), lambda k: (k, 0)),
                  _bspec((1, H), lambda k: (0, 0))],
        out_specs=_bspec((EBLK, H), lambda k: (k, 0)),
        out_shape=jax.ShapeDtypeStruct((EPAD, H), jnp.float32),
    )(xbs, xcd, b)


def _edge1(e, xbs, xcd, aw, ab):
    return pl.pallas_call(
        _edge1_body,
        grid=(EPAD // EBLK,),
        in_specs=[_bspec((EBLK, H), lambda k: (k, 0)),
                  _bspec((EBLK, H), lambda k: (k, 0)),
                  _bspec((EBLK, H), lambda k: (k, 0)),
                  _bspec((H, H), lambda k: (0, 0)),
                  _bspec((1, H), lambda k: (0, 0))],
        out_specs=_bspec((EBLK, H), lambda k: (k, 0)),
        out_shape=jax.ShapeDtypeStruct((EPAD, H), jnp.float32),
    )(e, xbs, xcd, aw, ab)


def _pool(e, xvs, xwd, wu, bu, wa, ba):
    return pl.pallas_call(
        _pool_body,
        grid=(NE // PBLK,),
        in_specs=[_bspec((PBLK, H), lambda k: (k, 0)),
                  _bspec((PBLK, H), lambda k: (k, 0)),
                  _bspec((PBLK, H), lambda k: (k, 0)),
                  _bspec((H, H), lambda k: (0, 0)),
                  _bspec((1, H), lambda k: (0, 0)),
                  _bspec((H, H), lambda k: (0, 0)),
                  _bspec((1, H), lambda k: (0, 0))],
        out_specs=_bspec((8, H), lambda k: (k, 0)),
        out_shape=jax.ShapeDtypeStruct((NMOL, H), jnp.float32),
    )(e, xvs, xwd, wu, bu, wa, ba)


def kernel(node_wid_list, node_child_adjacency_graph, node_edge_adjacency_graph,
           edge_node_adjacency_graph, scope, root_scope, params):
    p = params
    i32 = jnp.int32

    wid = node_wid_list.astype(i32)
    widp = jnp.concatenate([jnp.zeros((1,), i32), wid + 1,
                            jnp.zeros((NPAD - NN,), i32)])
    emb2 = jnp.concatenate([jnp.zeros((1, H), jnp.float32), p["embedding"]],
                           axis=0)
    src = edge_node_adjacency_graph[1:, 0].astype(i32)
    dst = edge_node_adjacency_graph[1:, 1].astype(i32)
    zpad = jnp.zeros((EPAD - NE,), i32)
    srcp = jnp.concatenate([src, zpad])
    dstp = jnp.concatenate([dst, zpad])
    ne_adj = node_edge_adjacency_graph.astype(i32)
    idx_g = jnp.where(ne_adj == 0, DEAD, ne_adj - 1)
    idx_g = jnp.concatenate(
        [idx_g, jnp.full((NPAD - NN, MAXN), DEAD, i32)], axis=0).T.reshape(-1)
    ch = node_child_adjacency_graph.astype(i32)
    idx_c = jnp.concatenate(
        [ch, jnp.zeros((NPAD - NN, MAXN), i32)], axis=0).T.reshape(-1)

    def cat_w(names, l=None):
        if l is None:
            return jnp.concatenate([p["pool_%s_w" % n] for n in names], axis=1)
        return jnp.concatenate([p["l%d_%s_w" % (l, n)] for n in names], axis=1)

    def cat_b(names, l=None):
        if l is None:
            v = jnp.concatenate([p["pool_%s_b" % n] for n in names])
        else:
            v = jnp.concatenate([p["l%d_%s_b" % (l, n)] for n in names])
        return v.reshape(1, -1)

    sc = _sc_kernels()

    x0 = sc["embed"](widp, emb2)

    xb0, xc0, vx0, xu0 = _proj(x0, cat_w("BCVU", 0), cat_b("BCVU", 0), 4)
    xbs0, xcd0 = sc["egather"](srcp, dstp, xb0, xc0)
    e1 = _edge0(xbs0, xcd0, p["l0_A_b"].reshape(1, H))
    ge0, wv0 = sc["ngather"](idx_g, idx_c, e1, vx0)

    xb1, xc1, vx1, xu1 = _node_update(ge0, wv0, xu0, cat_w("BCVU", 1),
                                      cat_b("BCVU", 1), 4)
    xbs1, xcd1 = sc["egather"](srcp, dstp, xb1, xc1)
    e2 = _edge1(e1, xbs1, xcd1, p["l1_A_w"], p["l1_A_b"].reshape(1, H))
    ge1, wv1 = sc["ngather"](idx_g, idx_c, e2, vx1)

    xv, xw = _node_update(ge1, wv1, xu1, cat_w("VW"), cat_b("VW"), 2)
    xvs, xwd = sc["egather"](srcp, dstp, xv, xw)

    return _pool(e2, xvs, xwd, p["pool_U_w"], p["pool_U_b"].reshape(1, H),
                 p["pool_A_w"], p["pool_A_b"].reshape(1, H))
